# baseline (speedup 1.0000x reference)
import jax
import jax.numpy as jnp
from jax import lax
from jax.experimental import pallas as pl
from jax.experimental.pallas import tpu as pltpu
from jax.experimental.pallas import tpu_sc as plsc

N = 100000
E = 3200000
H = 16

NC = 2
NS = 16
L = 16

N_PAD = 100352
CHUNK = 2048
RPC = CHUNK // 128
N_TILES = NC * NS
CHUNKS_PER_TILE = 50
PAIRS = CHUNKS_PER_TILE // 2
E_PAD = N_TILES * CHUNKS_PER_TILE * CHUNK
E_ROWS = E_PAD // 128


def _edge_pass(table_hbm, src_hbm, w_hbm, dst_hbm, m_hbm, part_hbm,
               big_v, a0, a1, b0, b1, mo0, mo1, sem_t, si0, si1, so0, so1):
    c = lax.axis_index("c")
    s = lax.axis_index("s")
    wid = c * NS + s
    row0 = wid * CHUNKS_PER_TILE * RPC

    avs = (a0, a1)
    bvs = (b0, b1)
    mos = (mo0, mo1)
    sis = (si0, si1)
    sos = (so0, so1)

    def rows(chunk_idx):
        return pl.ds(row0 + chunk_idx * RPC, RPC)

    def a_start(chunk_idx, p):
        pltpu.async_copy(src_hbm.at[rows(chunk_idx)], avs[p], sis[p])
        pltpu.async_copy(w_hbm.at[rows(chunk_idx)], bvs[p], sis[p])

    def a_wait(chunk_idx, p):
        pltpu.make_async_copy(src_hbm.at[rows(chunk_idx)], avs[p], sis[p]).wait()
        pltpu.make_async_copy(w_hbm.at[rows(chunk_idx)], bvs[p], sis[p]).wait()

    table_cp = pltpu.async_copy(table_hbm, big_v, sem_t)
    a_start(0, 0)
    a_start(1, 1)
    table_cp.wait()

    def gather(p):
        src_v, w_v, m_v = avs[p], bvs[p], mos[p]

        def g_body(j, _):
            for k in range(128 // L):
                idx = src_v[j, pl.ds(k * L, L)]
                vals = plsc.load_gather(big_v, [idx])
                m_v[j, pl.ds(k * L, L)] = vals * w_v[j, pl.ds(k * L, L)]
            return 0
        lax.fori_loop(0, RPC, g_body, 0)

    def pair_a(p, _):
        a = 2 * p
        a_wait(a, 0)
        gather(0)
        wr0 = pltpu.async_copy(mo0, m_hbm.at[rows(a)], so0)
        a_wait(a + 1, 1)
        gather(1)
        wr1 = pltpu.async_copy(mo1, m_hbm.at[rows(a + 1)], so1)

        @pl.when(p < PAIRS - 1)
        def _():
            a_start(a + 2, 0)
            a_start(a + 3, 1)
        wr0.wait()
        wr1.wait()
        return 0

    lax.fori_loop(0, PAIRS, pair_a, 0)

    def b_start(chunk_idx, p):
        pltpu.async_copy(dst_hbm.at[rows(chunk_idx)], avs[p], sis[p])
        pltpu.async_copy(m_hbm.at[rows(chunk_idx)], bvs[p], sis[p])

    def b_wait(chunk_idx, p):
        pltpu.make_async_copy(dst_hbm.at[rows(chunk_idx)], avs[p], sis[p]).wait()
        pltpu.make_async_copy(m_hbm.at[rows(chunk_idx)], bvs[p], sis[p]).wait()

    b_start(0, 0)
    b_start(1, 1)

    def zero_body(i, _):
        big_v[pl.ds(i * L, L)] = jnp.zeros((L,), jnp.float32)
        return 0
    lax.fori_loop(0, N_PAD // L, zero_body, 0)

    def scat(p):
        dst_v, m_v = avs[p], bvs[p]

        def s_body(j, _):
            for k in range(128 // L):
                idx = dst_v[j, pl.ds(k * L, L)]
                mv = m_v[j, pl.ds(k * L, L)]
                plsc.addupdate_scatter(big_v, [idx], mv)
            return 0
        lax.fori_loop(0, RPC, s_body, 0)

    def pair_b(p, _):
        a = 2 * p
        b_wait(a, 0)
        scat(0)

        @pl.when(p < PAIRS - 1)
        def _():
            b_start(a + 2, 0)
        b_wait(a + 1, 1)
        scat(1)

        @pl.when(p < PAIRS - 1)
        def _():
            b_start(a + 3, 1)
        return 0

    lax.fori_loop(0, PAIRS, pair_b, 0)

    pltpu.sync_copy(big_v, part_hbm.at[wid])


def _make_edge_pass(interpret=False):
    mesh = plsc.VectorSubcoreMesh(core_axis_name="c", subcore_axis_name="s",
                                  num_cores=NC, num_subcores=NS)
    r = RPC
    return pl.kernel(
        _edge_pass,
        out_type=(
            jax.ShapeDtypeStruct((E_ROWS, 128), jnp.float32),
            jax.ShapeDtypeStruct((N_TILES, N_PAD), jnp.float32),
        ),
        mesh=mesh,
        scratch_types=[
            pltpu.VMEM((N_PAD,), jnp.float32),
            pltpu.VMEM((r, 128), jnp.int32),
            pltpu.VMEM((r, 128), jnp.int32),
            pltpu.VMEM((r, 128), jnp.float32),
            pltpu.VMEM((r, 128), jnp.float32),
            pltpu.VMEM((r, 128), jnp.float32),
            pltpu.VMEM((r, 128), jnp.float32),
            pltpu.SemaphoreType.DMA,
            pltpu.SemaphoreType.DMA,
            pltpu.SemaphoreType.DMA,
            pltpu.SemaphoreType.DMA,
            pltpu.SemaphoreType.DMA,
        ],
        compiler_params=pltpu.CompilerParams(needs_layout_passes=False),
        interpret=interpret,
    )


_ROWS = N_PAD // 128
_BLK = 112


def _reduce_node_stage(w_ref, p_ref, x_ref, t_ref, r_ref):
    sv = jnp.sum(p_ref[...], axis=0)
    xv = x_ref[...]
    t = jnp.zeros_like(sv)
    r = jnp.zeros_like(sv)
    for k in range(H):
        h = jnp.maximum(sv * w_ref[0, k] + xv * w_ref[2, k] + w_ref[1, k], 0.0)
        t = t + h * w_ref[3, k]
        r = r + h * w_ref[4, k]
    t_ref[...] = t
    r_ref[...] = r + w_ref[5, 0]


def _node_kernel(wmat, parts3d, x2d, interpret=False):
    return pl.pallas_call(
        _reduce_node_stage,
        grid=(_ROWS // _BLK,),
        in_specs=[
            pl.BlockSpec(memory_space=pltpu.SMEM),
            pl.BlockSpec((N_TILES, _BLK, 128), lambda i: (0, i, 0)),
            pl.BlockSpec((_BLK, 128), lambda i: (i, 0)),
        ],
        out_specs=[
            pl.BlockSpec((_BLK, 128), lambda i: (i, 0)),
            pl.BlockSpec((_BLK, 128), lambda i: (i, 0)),
        ],
        out_shape=[
            jax.ShapeDtypeStruct((_ROWS, 128), jnp.float32),
            jax.ShapeDtypeStruct((_ROWS, 128), jnp.float32),
        ],
        interpret=interpret,
    )(wmat, parts3d, x2d)


def _reduce_add(p_ref, r_ref, o_ref):
    o_ref[...] = jnp.sum(p_ref[...], axis=0) + r_ref[...]


def _final_kernel(parts3d, r2d, interpret=False):
    return pl.pallas_call(
        _reduce_add,
        grid=(_ROWS // _BLK,),
        in_specs=[
            pl.BlockSpec((N_TILES, _BLK, 128), lambda i: (0, i, 0)),
            pl.BlockSpec((_BLK, 128), lambda i: (i, 0)),
        ],
        out_specs=pl.BlockSpec((_BLK, 128), lambda i: (i, 0)),
        out_shape=jax.ShapeDtypeStruct((_ROWS, 128), jnp.float32),
        interpret=interpret,
    )(parts3d, r2d)


@jax.jit
def _run(x, edge_index, edge_weight,
         W1_rel, b1_rel, W1_root, W2_rel, b2_rel, W2_root):
    src = edge_index[0]
    dst = edge_index[1]
    pad = E_PAD - E
    src_p = jnp.concatenate([src, jnp.zeros((pad,), jnp.int32)])
    dst_p = jnp.concatenate([dst, jnp.full((pad,), N, jnp.int32)])
    w_p = jnp.concatenate([edge_weight, jnp.zeros((pad,), jnp.float32)])
    src2d = src_p.reshape(E_ROWS, 128)
    dst2d = dst_p.reshape(E_ROWS, 128)
    w2d = w_p.reshape(E_ROWS, 128)

    x_flat = x.reshape(-1)
    x_pad = jnp.concatenate([x_flat, jnp.zeros((N_PAD - N,), jnp.float32)])

    edge_pass = _make_edge_pass()

    _, s_parts = edge_pass(x_pad, src2d, w2d, dst2d)

    wmat = jnp.stack([
        W1_rel[:, 0], b1_rel, W1_root[:, 0],
        W2_rel[0, :], W2_root[0, :],
        jnp.full((H,), b2_rel[0], jnp.float32),
    ])
    t2d, r2d = _node_kernel(wmat, s_parts.reshape(N_TILES, _ROWS, 128),
                            x_pad.reshape(_ROWS, 128))
    t_pad = t2d.reshape(-1)

    _, u_parts = edge_pass(t_pad, src2d, w2d, dst2d)
    out2d = _final_kernel(u_parts.reshape(N_TILES, _ROWS, 128), r2d)
    return out2d.reshape(-1)[:N]


def kernel(x, edge_index, edge_weight,
           W1_rel, b1_rel, W1_root, W2_rel, b2_rel, W2_root):
    return _run(x, edge_index, edge_weight,
                W1_rel, b1_rel, W1_root, W2_rel, b2_rel, W2_root)


# speedup vs baseline: 1.1944x; 1.1944x over previous
import jax
import jax.numpy as jnp
from jax import lax
from jax.experimental import pallas as pl
from jax.experimental.pallas import tpu as pltpu
from jax.experimental.pallas import tpu_sc as plsc

N = 100000
E = 3200000
H = 16

NC = 2
NS = 16
L = 16

N_PAD = 100352
CHUNK = 2048
RPC = CHUNK // 128
N_TILES = NC * NS
CHUNKS_PER_TILE = 50
PAIRS = CHUNKS_PER_TILE // 2
E_PAD = N_TILES * CHUNKS_PER_TILE * CHUNK
E_ROWS = E_PAD // 128


NSLOT = 4
MAIN_TRIPS = CHUNKS_PER_TILE // NSLOT - 1
TAIL = CHUNKS_PER_TILE - NSLOT * (MAIN_TRIPS + 1)


def _edge_pass(table_hbm, src_hbm, w_hbm, dst_hbm, m_hbm, part_hbm,
               big_v, a0, a1, a2, a3, b0, b1, b2, b3, mo0, mo1, mo2, mo3,
               sem_t, si0, si1, si2, si3, so0, so1, so2, so3):
    c = lax.axis_index("c")
    s = lax.axis_index("s")
    wid = c * NS + s
    row0 = wid * CHUNKS_PER_TILE * RPC

    avs = (a0, a1, a2, a3)
    bvs = (b0, b1, b2, b3)
    mos = (mo0, mo1, mo2, mo3)
    sis = (si0, si1, si2, si3)
    sos = (so0, so1, so2, so3)

    def rows(chunk_idx):
        return pl.ds(row0 + chunk_idx * RPC, RPC)

    def a_start(chunk_idx, j):
        pltpu.async_copy(src_hbm.at[rows(chunk_idx)], avs[j], sis[j])
        pltpu.async_copy(w_hbm.at[rows(chunk_idx)], bvs[j], sis[j])

    def a_wait(chunk_idx, j):
        pltpu.make_async_copy(src_hbm.at[rows(chunk_idx)], avs[j], sis[j]).wait()
        pltpu.make_async_copy(w_hbm.at[rows(chunk_idx)], bvs[j], sis[j]).wait()

    def wr_wait(chunk_idx, j):
        pltpu.make_async_copy(mos[j], m_hbm.at[rows(chunk_idx)], sos[j]).wait()

    table_cp = pltpu.async_copy(table_hbm, big_v, sem_t)
    for j in range(NSLOT):
        a_start(j, j)
    table_cp.wait()

    def gather(j):
        src_v, w_v, m_v = avs[j], bvs[j], mos[j]

        def g_body(i, _):
            for k in range(128 // L):
                idx = src_v[i, pl.ds(k * L, L)]
                vals = plsc.load_gather(big_v, [idx])
                m_v[i, pl.ds(k * L, L)] = vals * w_v[i, pl.ds(k * L, L)]
            return 0
        lax.fori_loop(0, RPC, g_body, 0)

    def loop_a(p, _):
        base = NSLOT * p
        for j in range(NSLOT):
            a_wait(base + j, j)

            @pl.when(p > 0)
            def _():
                wr_wait(base + j - NSLOT, j)
            gather(j)
            pltpu.async_copy(mos[j], m_hbm.at[rows(base + j)], sos[j])

            @pl.when(base + j + NSLOT < CHUNKS_PER_TILE)
            def _():
                a_start(base + j + NSLOT, j)
        return 0

    lax.fori_loop(0, MAIN_TRIPS + 1, loop_a, 0)
    for j in range(TAIL):
        cidx = NSLOT * (MAIN_TRIPS + 1) + j
        a_wait(cidx, j)
        wr_wait(cidx - NSLOT, j)
        gather(j)
        pltpu.async_copy(mos[j], m_hbm.at[rows(cidx)], sos[j])
    for j in range(TAIL, NSLOT):
        wr_wait(NSLOT * MAIN_TRIPS + j, j)
    for j in range(TAIL):
        wr_wait(NSLOT * (MAIN_TRIPS + 1) + j, j)

    def b_start(chunk_idx, j):
        pltpu.async_copy(dst_hbm.at[rows(chunk_idx)], avs[j], sis[j])
        pltpu.async_copy(m_hbm.at[rows(chunk_idx)], bvs[j], sis[j])

    def b_wait(chunk_idx, j):
        pltpu.make_async_copy(dst_hbm.at[rows(chunk_idx)], avs[j], sis[j]).wait()
        pltpu.make_async_copy(m_hbm.at[rows(chunk_idx)], bvs[j], sis[j]).wait()

    for j in range(NSLOT):
        b_start(j, j)

    def zero_body(i, _):
        for k in range(8):
            big_v[pl.ds(i * 8 * L + k * L, L)] = jnp.zeros((L,), jnp.float32)
        return 0
    lax.fori_loop(0, N_PAD // (8 * L), zero_body, 0)

    def scat(j):
        dst_v, m_v = avs[j], bvs[j]

        def s_body(i, _):
            for k in range(128 // L):
                idx = dst_v[i, pl.ds(k * L, L)]
                mv = m_v[i, pl.ds(k * L, L)]
                plsc.addupdate_scatter(big_v, [idx], mv)
            return 0
        lax.fori_loop(0, RPC, s_body, 0)

    def loop_b(p, _):
        base = NSLOT * p
        for j in range(NSLOT):
            b_wait(base + j, j)
            scat(j)

            @pl.when(base + j + NSLOT < CHUNKS_PER_TILE)
            def _():
                b_start(base + j + NSLOT, j)
        return 0

    lax.fori_loop(0, MAIN_TRIPS + 1, loop_b, 0)
    for j in range(TAIL):
        cidx = NSLOT * (MAIN_TRIPS + 1) + j
        b_wait(cidx, j)
        scat(j)

    pltpu.sync_copy(big_v, part_hbm.at[wid])


def _make_edge_pass(interpret=False):
    mesh = plsc.VectorSubcoreMesh(core_axis_name="c", subcore_axis_name="s",
                                  num_cores=NC, num_subcores=NS)
    r = RPC
    return pl.kernel(
        _edge_pass,
        out_type=(
            jax.ShapeDtypeStruct((E_ROWS, 128), jnp.float32),
            jax.ShapeDtypeStruct((N_TILES, N_PAD), jnp.float32),
        ),
        mesh=mesh,
        scratch_types=(
            [pltpu.VMEM((N_PAD,), jnp.float32)]
            + [pltpu.VMEM((r, 128), jnp.int32)] * 4
            + [pltpu.VMEM((r, 128), jnp.float32)] * 4
            + [pltpu.VMEM((r, 128), jnp.float32)] * 4
            + [pltpu.SemaphoreType.DMA] * 9
        ),
        compiler_params=pltpu.CompilerParams(needs_layout_passes=False),
        interpret=interpret,
    )


_ROWS = N_PAD // 128
_BLK = 112


def _reduce_node_stage(w_ref, p_ref, x_ref, t_ref, r_ref):
    sv = jnp.sum(p_ref[...], axis=0)
    xv = x_ref[...]
    t = jnp.zeros_like(sv)
    r = jnp.zeros_like(sv)
    for k in range(H):
        h = jnp.maximum(sv * w_ref[0, k] + xv * w_ref[2, k] + w_ref[1, k], 0.0)
        t = t + h * w_ref[3, k]
        r = r + h * w_ref[4, k]
    t_ref[...] = t
    r_ref[...] = r + w_ref[5, 0]


def _node_kernel(wmat, parts3d, x2d, interpret=False):
    return pl.pallas_call(
        _reduce_node_stage,
        grid=(_ROWS // _BLK,),
        in_specs=[
            pl.BlockSpec(memory_space=pltpu.SMEM),
            pl.BlockSpec((N_TILES, _BLK, 128), lambda i: (0, i, 0)),
            pl.BlockSpec((_BLK, 128), lambda i: (i, 0)),
        ],
        out_specs=[
            pl.BlockSpec((_BLK, 128), lambda i: (i, 0)),
            pl.BlockSpec((_BLK, 128), lambda i: (i, 0)),
        ],
        out_shape=[
            jax.ShapeDtypeStruct((_ROWS, 128), jnp.float32),
            jax.ShapeDtypeStruct((_ROWS, 128), jnp.float32),
        ],
        interpret=interpret,
    )(wmat, parts3d, x2d)


def _reduce_add(p_ref, r_ref, o_ref):
    o_ref[...] = jnp.sum(p_ref[...], axis=0) + r_ref[...]


def _final_kernel(parts3d, r2d, interpret=False):
    return pl.pallas_call(
        _reduce_add,
        grid=(_ROWS // _BLK,),
        in_specs=[
            pl.BlockSpec((N_TILES, _BLK, 128), lambda i: (0, i, 0)),
            pl.BlockSpec((_BLK, 128), lambda i: (i, 0)),
        ],
        out_specs=pl.BlockSpec((_BLK, 128), lambda i: (i, 0)),
        out_shape=jax.ShapeDtypeStruct((_ROWS, 128), jnp.float32),
        interpret=interpret,
    )(parts3d, r2d)


@jax.jit
def _run(x, edge_index, edge_weight,
         W1_rel, b1_rel, W1_root, W2_rel, b2_rel, W2_root):
    src = edge_index[0]
    dst = edge_index[1]
    pad = E_PAD - E
    src_p = jnp.concatenate([src, jnp.zeros((pad,), jnp.int32)])
    dst_p = jnp.concatenate([dst, jnp.full((pad,), N, jnp.int32)])
    w_p = jnp.concatenate([edge_weight, jnp.zeros((pad,), jnp.float32)])
    src2d = src_p.reshape(E_ROWS, 128)
    dst2d = dst_p.reshape(E_ROWS, 128)
    w2d = w_p.reshape(E_ROWS, 128)

    x_flat = x.reshape(-1)
    x_pad = jnp.concatenate([x_flat, jnp.zeros((N_PAD - N,), jnp.float32)])

    edge_pass = _make_edge_pass()

    _, s_parts = edge_pass(x_pad, src2d, w2d, dst2d)

    wmat = jnp.stack([
        W1_rel[:, 0], b1_rel, W1_root[:, 0],
        W2_rel[0, :], W2_root[0, :],
        jnp.full((H,), b2_rel[0], jnp.float32),
    ])
    t2d, r2d = _node_kernel(wmat, s_parts.reshape(N_TILES, _ROWS, 128),
                            x_pad.reshape(_ROWS, 128))
    t_pad = t2d.reshape(-1)

    _, u_parts = edge_pass(t_pad, src2d, w2d, dst2d)
    out2d = _final_kernel(u_parts.reshape(N_TILES, _ROWS, 128), r2d)
    return out2d.reshape(-1)[:N]


def kernel(x, edge_index, edge_weight,
           W1_rel, b1_rel, W1_root, W2_rel, b2_rel, W2_root):
    return _run(x, edge_index, edge_weight,
                W1_rel, b1_rel, W1_root, W2_rel, b2_rel, W2_root)


# speedup vs baseline: 1.8135x; 1.5183x over previous
import jax
import jax.numpy as jnp
from jax import lax
from jax.experimental import pallas as pl
from jax.experimental.pallas import tpu as pltpu
from jax.experimental.pallas import tpu_sc as plsc

N = 100000
E = 3200000
H = 16

NC = 2
NS = 16
L = 16

N_PAD = 100352
CHUNK = 2048
RPC = CHUNK // 128
N_TILES = NC * NS
NSLOT = 4

FULL_CHUNKS = 1562
BIG_TILES = 26
MAIN_TRIPS = 12
TAIL_OFF = FULL_CHUNKS * CHUNK
TAIL_LEN = E - TAIL_OFF


def _edge_pass(table_hbm, src_hbm, w_hbm, dst_hbm, m_hbm, part_hbm,
               big_v, a0, a1, a2, a3, b0, b1, b2, b3, mo0, mo1, mo2, mo3,
               sem_t, si0, si1, si2, si3, so0, so1, so2, so3):
    c = lax.axis_index("c")
    s = lax.axis_index("s")
    wid = c * NS + s
    nfull = jnp.where(wid < BIG_TILES, 49, 48)
    base_el = CHUNK * (48 * wid + jnp.minimum(wid, BIG_TILES))

    avs = (a0, a1, a2, a3)
    bvs = (b0, b1, b2, b3)
    mos = (mo0, mo1, mo2, mo3)
    sis = (si0, si1, si2, si3)
    sos = (so0, so1, so2, so3)

    def sl(chunk_idx):
        return pl.ds(base_el + chunk_idx * CHUNK, CHUNK)

    def a_start(chunk_idx, j):
        pltpu.async_copy(src_hbm.at[sl(chunk_idx)], avs[j], sis[j])
        pltpu.async_copy(w_hbm.at[sl(chunk_idx)], bvs[j], sis[j])

    def a_wait(chunk_idx, j):
        pltpu.make_async_copy(src_hbm.at[sl(chunk_idx)], avs[j], sis[j]).wait()
        pltpu.make_async_copy(w_hbm.at[sl(chunk_idx)], bvs[j], sis[j]).wait()

    def wr_wait(chunk_idx, j):
        pltpu.make_async_copy(mos[j], m_hbm.at[sl(chunk_idx)], sos[j]).wait()

    with jax.named_scope("prep"):
        table_cp = pltpu.async_copy(table_hbm, big_v.at[pl.ds(0, N)], sem_t)
        for j in range(NSLOT):
            a_start(j, j)
        table_cp.wait()

    def gather(j, nrows=RPC):
        src_v, w_v, m_v = avs[j], bvs[j], mos[j]

        def g_body(i, _):
            for k in range(128 // L):
                q = pl.ds(i * 128 + k * L, L)
                idx = src_v[q]
                vals = plsc.load_gather(big_v, [idx])
                m_v[q] = vals * w_v[q]
            return 0
        lax.fori_loop(0, nrows, g_body, 0)

    def loop_a(p, _):
        base = NSLOT * p
        for j in range(NSLOT):
            a_wait(base + j, j)

            @pl.when(p > 0)
            def _():
                wr_wait(base + j - NSLOT, j)
            gather(j)
            pltpu.async_copy(mos[j], m_hbm.at[sl(base + j)], sos[j])

            @pl.when(base + j + NSLOT < nfull)
            def _():
                a_start(base + j + NSLOT, j)
        return 0

    with jax.named_scope("phaseA"):
        lax.fori_loop(0, MAIN_TRIPS, loop_a, 0)

        @pl.when(nfull == 49)
        def _():
            a_wait(48, 0)
            wr_wait(44, 0)
            gather(0)
            pltpu.async_copy(mos[0], m_hbm.at[sl(48)], so0)
        cidx0 = jnp.where(nfull == 49, 48, 44)
        wr_wait(cidx0, 0)
        for j in range(1, NSLOT):
            wr_wait(44 + j, j)

        @pl.when(wid == N_TILES - 1)
        def _():
            tsl = pl.ds(TAIL_OFF, TAIL_LEN)
            pltpu.sync_copy(src_hbm.at[tsl], a0.at[pl.ds(0, TAIL_LEN)])
            pltpu.sync_copy(w_hbm.at[tsl], b0.at[pl.ds(0, TAIL_LEN)])
            gather(0, nrows=TAIL_LEN // 128)
            pltpu.sync_copy(mo0.at[pl.ds(0, TAIL_LEN)], m_hbm.at[tsl])

    def b_start(chunk_idx, j):
        pltpu.async_copy(dst_hbm.at[sl(chunk_idx)], avs[j], sis[j])
        pltpu.async_copy(m_hbm.at[sl(chunk_idx)], bvs[j], sis[j])

    def b_wait(chunk_idx, j):
        pltpu.make_async_copy(dst_hbm.at[sl(chunk_idx)], avs[j], sis[j]).wait()
        pltpu.make_async_copy(m_hbm.at[sl(chunk_idx)], bvs[j], sis[j]).wait()

    def scat(j, nrows=RPC):
        dst_v, m_v = avs[j], bvs[j]

        def s_body(i, _):
            for k in range(128 // L):
                q = pl.ds(i * 128 + k * L, L)
                plsc.addupdate_scatter(big_v, [dst_v[q]], m_v[q])
            return 0
        lax.fori_loop(0, nrows, s_body, 0)

    with jax.named_scope("zero"):
        for j in range(NSLOT):
            b_start(j, j)

        def zero_body(i, _):
            for k in range(8):
                big_v[pl.ds(i * 8 * L + k * L, L)] = jnp.zeros((L,), jnp.float32)
            return 0
        lax.fori_loop(0, N_PAD // (8 * L), zero_body, 0)

    def loop_b(p, _):
        base = NSLOT * p
        for j in range(NSLOT):
            b_wait(base + j, j)
            scat(j)

            @pl.when(base + j + NSLOT < nfull)
            def _():
                b_start(base + j + NSLOT, j)
        return 0

    with jax.named_scope("phaseB"):
        lax.fori_loop(0, MAIN_TRIPS, loop_b, 0)

        @pl.when(nfull == 49)
        def _():
            b_wait(48, 0)
            scat(0)

        @pl.when(wid == N_TILES - 1)
        def _():
            tsl = pl.ds(TAIL_OFF, TAIL_LEN)
            pltpu.sync_copy(dst_hbm.at[tsl], a0.at[pl.ds(0, TAIL_LEN)])
            pltpu.sync_copy(m_hbm.at[tsl], b0.at[pl.ds(0, TAIL_LEN)])
            scat(0, nrows=TAIL_LEN // 128)

    with jax.named_scope("writeout"):
        pltpu.sync_copy(big_v, part_hbm.at[wid])


def _make_edge_pass(interpret=False):
    mesh = plsc.VectorSubcoreMesh(core_axis_name="c", subcore_axis_name="s",
                                  num_cores=NC, num_subcores=NS)
    return pl.kernel(
        _edge_pass,
        out_type=(
            jax.ShapeDtypeStruct((E,), jnp.float32),
            jax.ShapeDtypeStruct((N_TILES, N_PAD), jnp.float32),
        ),
        mesh=mesh,
        scratch_types=(
            [pltpu.VMEM((N_PAD,), jnp.float32)]
            + [pltpu.VMEM((CHUNK,), jnp.int32)] * 4
            + [pltpu.VMEM((CHUNK,), jnp.float32)] * 4
            + [pltpu.VMEM((CHUNK,), jnp.float32)] * 4
            + [pltpu.SemaphoreType.DMA] * 9
        ),
        compiler_params=pltpu.CompilerParams(needs_layout_passes=False),
        interpret=interpret,
    )


_ROWS = N_PAD // 128
_BLK = 112


def _reduce_node_stage(w_ref, p_ref, x_ref, t_ref, r_ref):
    sv = jnp.sum(p_ref[...], axis=0)
    xv = x_ref[...]
    t = jnp.zeros_like(sv)
    r = jnp.zeros_like(sv)
    for k in range(H):
        h = jnp.maximum(sv * w_ref[0, k] + xv * w_ref[2, k] + w_ref[1, k], 0.0)
        t = t + h * w_ref[3, k]
        r = r + h * w_ref[4, k]
    t_ref[...] = t
    r_ref[...] = r + w_ref[5, 0]


def _node_kernel(wmat, parts3d, x2d, interpret=False):
    return pl.pallas_call(
        _reduce_node_stage,
        grid=(_ROWS // _BLK,),
        in_specs=[
            pl.BlockSpec(memory_space=pltpu.SMEM),
            pl.BlockSpec((N_TILES, _BLK, 128), lambda i: (0, i, 0)),
            pl.BlockSpec((_BLK, 128), lambda i: (i, 0)),
        ],
        out_specs=[
            pl.BlockSpec((_BLK, 128), lambda i: (i, 0)),
            pl.BlockSpec((_BLK, 128), lambda i: (i, 0)),
        ],
        out_shape=[
            jax.ShapeDtypeStruct((_ROWS, 128), jnp.float32),
            jax.ShapeDtypeStruct((_ROWS, 128), jnp.float32),
        ],
        interpret=interpret,
    )(wmat, parts3d, x2d)


def _reduce_add(p_ref, r_ref, o_ref):
    o_ref[...] = jnp.sum(p_ref[...], axis=0) + r_ref[...]


def _final_kernel(parts3d, r2d, interpret=False):
    return pl.pallas_call(
        _reduce_add,
        grid=(_ROWS // _BLK,),
        in_specs=[
            pl.BlockSpec((N_TILES, _BLK, 128), lambda i: (0, i, 0)),
            pl.BlockSpec((_BLK, 128), lambda i: (i, 0)),
        ],
        out_specs=pl.BlockSpec((_BLK, 128), lambda i: (i, 0)),
        out_shape=jax.ShapeDtypeStruct((_ROWS, 128), jnp.float32),
        interpret=interpret,
    )(parts3d, r2d)


@jax.jit
def _run(x, edge_index, edge_weight,
         W1_rel, b1_rel, W1_root, W2_rel, b2_rel, W2_root):
    src = edge_index[0]
    dst = edge_index[1]
    x_flat = x.reshape(-1)
    x_pad2d = jnp.concatenate(
        [x_flat, jnp.zeros((N_PAD - N,), jnp.float32)]).reshape(_ROWS, 128)

    edge_pass = _make_edge_pass()

    _, s_parts = edge_pass(x_flat, src, edge_weight, dst)

    wmat = jnp.stack([
        W1_rel[:, 0], b1_rel, W1_root[:, 0],
        W2_rel[0, :], W2_root[0, :],
        jnp.full((H,), b2_rel[0], jnp.float32),
    ])
    t2d, r2d = _node_kernel(wmat, s_parts.reshape(N_TILES, _ROWS, 128),
                            x_pad2d)
    t_flat = t2d.reshape(-1)[:N]

    _, u_parts = edge_pass(t_flat, src, edge_weight, dst)
    out2d = _final_kernel(u_parts.reshape(N_TILES, _ROWS, 128), r2d)
    return out2d.reshape(-1)[:N]


def kernel(x, edge_index, edge_weight,
           W1_rel, b1_rel, W1_root, W2_rel, b2_rel, W2_root):
    return _run(x, edge_index, edge_weight,
                W1_rel, b1_rel, W1_root, W2_rel, b2_rel, W2_root)


# speedup vs baseline: 2.4287x; 1.3393x over previous
import jax
import jax.numpy as jnp
from jax import lax
from jax.experimental import pallas as pl
from jax.experimental.pallas import tpu as pltpu
from jax.experimental.pallas import tpu_sc as plsc

N = 100000
E = 3200000
H = 16

NC = 2
NS = 16
L = 16

N_PAD = 100352
CHUNK = 2048
RPC = CHUNK // 128
N_TILES = NC * NS
NSLOT = 4

FULL_CHUNKS = 1562
BIG_TILES = 26
MAIN_TRIPS = 12
TAIL_OFF = FULL_CHUNKS * CHUNK
TAIL_LEN = E - TAIL_OFF


def _edge_pass(table_hbm, ei_hbm, w_hbm, m_hbm, part_hbm,
               big_v, a0, a1, a2, a3, b0, b1, b2, b3, mo0, mo1, mo2, mo3,
               sem_t, si0, si1, si2, si3, so0, so1, so2, so3):
    src_hbm = ei_hbm.at[0]
    dst_hbm = ei_hbm.at[1]
    c = lax.axis_index("c")
    s = lax.axis_index("s")
    wid = c * NS + s
    nfull = jnp.where(wid < BIG_TILES, 49, 48)
    base_el = CHUNK * (48 * wid + jnp.minimum(wid, BIG_TILES))

    avs = (a0, a1, a2, a3)
    bvs = (b0, b1, b2, b3)
    mos = (mo0, mo1, mo2, mo3)
    sis = (si0, si1, si2, si3)
    sos = (so0, so1, so2, so3)

    def sl(chunk_idx):
        return pl.ds(base_el + chunk_idx * CHUNK, CHUNK)

    def a_start(chunk_idx, j):
        pltpu.async_copy(src_hbm.at[sl(chunk_idx)], avs[j], sis[j])
        pltpu.async_copy(w_hbm.at[sl(chunk_idx)], bvs[j], sis[j])

    def a_wait(chunk_idx, j):
        pltpu.make_async_copy(src_hbm.at[sl(chunk_idx)], avs[j], sis[j]).wait()
        pltpu.make_async_copy(w_hbm.at[sl(chunk_idx)], bvs[j], sis[j]).wait()

    def wr_wait(chunk_idx, j):
        pltpu.make_async_copy(mos[j], m_hbm.at[sl(chunk_idx)], sos[j]).wait()

    with jax.named_scope("prep"):
        table_cp = pltpu.async_copy(table_hbm, big_v.at[pl.ds(0, N)], sem_t)
        for j in range(NSLOT):
            a_start(j, j)
        table_cp.wait()

    def gather(j, nrows=RPC):
        src_v, w_v, m_v = avs[j], bvs[j], mos[j]

        def g_body(i, _):
            for k in range(128 // L):
                q = pl.ds(i * 128 + k * L, L)
                idx = src_v[q]
                vals = plsc.load_gather(big_v, [idx])
                m_v[q] = vals * w_v[q]
            return 0
        lax.fori_loop(0, nrows, g_body, 0)

    def loop_a(p, _):
        base = NSLOT * p
        for j in range(NSLOT):
            a_wait(base + j, j)

            @pl.when(p > 0)
            def _():
                wr_wait(base + j - NSLOT, j)
            gather(j)
            pltpu.async_copy(mos[j], m_hbm.at[sl(base + j)], sos[j])

            @pl.when(base + j + NSLOT < nfull)
            def _():
                a_start(base + j + NSLOT, j)
        return 0

    with jax.named_scope("phaseA"):
        lax.fori_loop(0, MAIN_TRIPS, loop_a, 0)

        @pl.when(nfull == 49)
        def _():
            a_wait(48, 0)
            wr_wait(44, 0)
            gather(0)
            pltpu.async_copy(mos[0], m_hbm.at[sl(48)], so0)
        cidx0 = jnp.where(nfull == 49, 48, 44)
        wr_wait(cidx0, 0)
        for j in range(1, NSLOT):
            wr_wait(44 + j, j)

        @pl.when(wid == N_TILES - 1)
        def _():
            tsl = pl.ds(TAIL_OFF, TAIL_LEN)
            pltpu.sync_copy(src_hbm.at[tsl], a0.at[pl.ds(0, TAIL_LEN)])
            pltpu.sync_copy(w_hbm.at[tsl], b0.at[pl.ds(0, TAIL_LEN)])
            gather(0, nrows=TAIL_LEN // 128)
            pltpu.sync_copy(mo0.at[pl.ds(0, TAIL_LEN)], m_hbm.at[tsl])

    def b_start(chunk_idx, j):
        pltpu.async_copy(dst_hbm.at[sl(chunk_idx)], avs[j], sis[j])
        pltpu.async_copy(m_hbm.at[sl(chunk_idx)], bvs[j], sis[j])

    def b_wait(chunk_idx, j):
        pltpu.make_async_copy(dst_hbm.at[sl(chunk_idx)], avs[j], sis[j]).wait()
        pltpu.make_async_copy(m_hbm.at[sl(chunk_idx)], bvs[j], sis[j]).wait()

    def scat(j, nrows=RPC):
        dst_v, m_v = avs[j], bvs[j]

        @plsc.parallel_loop(0, nrows * 128, step=L, unroll=4)
        def _(q0):
            q = pl.ds(q0, L)
            plsc.addupdate_scatter(big_v, [dst_v[q]], m_v[q])

    with jax.named_scope("zero"):
        for j in range(NSLOT):
            b_start(j, j)

        def zero_body(i, _):
            for k in range(8):
                big_v[pl.ds(i * 8 * L + k * L, L)] = jnp.zeros((L,), jnp.float32)
            return 0
        lax.fori_loop(0, N_PAD // (8 * L), zero_body, 0)

    def loop_b(p, _):
        base = NSLOT * p
        for j in range(NSLOT):
            b_wait(base + j, j)
            scat(j)

            @pl.when(base + j + NSLOT < nfull)
            def _():
                b_start(base + j + NSLOT, j)
        return 0

    with jax.named_scope("phaseB"):
        lax.fori_loop(0, MAIN_TRIPS, loop_b, 0)

        @pl.when(nfull == 49)
        def _():
            b_wait(48, 0)
            scat(0)

        @pl.when(wid == N_TILES - 1)
        def _():
            tsl = pl.ds(TAIL_OFF, TAIL_LEN)
            pltpu.sync_copy(dst_hbm.at[tsl], a0.at[pl.ds(0, TAIL_LEN)])
            pltpu.sync_copy(m_hbm.at[tsl], b0.at[pl.ds(0, TAIL_LEN)])
            scat(0, nrows=TAIL_LEN // 128)

    with jax.named_scope("writeout"):
        pltpu.sync_copy(big_v, part_hbm.at[wid])


def _make_edge_pass(interpret=False):
    mesh = plsc.VectorSubcoreMesh(core_axis_name="c", subcore_axis_name="s",
                                  num_cores=NC, num_subcores=NS)
    return pl.kernel(
        _edge_pass,
        out_type=(
            jax.ShapeDtypeStruct((E,), jnp.float32),
            jax.ShapeDtypeStruct((N_TILES, N_PAD), jnp.float32),
        ),
        name="edge_pass",
        mesh=mesh,
        scratch_types=(
            [pltpu.VMEM((N_PAD,), jnp.float32)]
            + [pltpu.VMEM((CHUNK,), jnp.int32)] * 4
            + [pltpu.VMEM((CHUNK,), jnp.float32)] * 4
            + [pltpu.VMEM((CHUNK,), jnp.float32)] * 4
            + [pltpu.SemaphoreType.DMA] * 9
        ),
        compiler_params=pltpu.CompilerParams(needs_layout_passes=False),
        interpret=interpret,
    )


_ROWS = N_PAD // 128
_BLK = 112


def _reduce_node_stage(w_ref, p_ref, x_ref, t_ref, r_ref):
    sv = jnp.sum(p_ref[...], axis=0)
    xv = x_ref[...]
    t = jnp.zeros_like(sv)
    r = jnp.zeros_like(sv)
    for k in range(H):
        h = jnp.maximum(sv * w_ref[0, k] + xv * w_ref[2, k] + w_ref[1, k], 0.0)
        t = t + h * w_ref[3, k]
        r = r + h * w_ref[4, k]
    t_ref[...] = t
    r_ref[...] = r + w_ref[5, 0]


def _node_kernel(wmat, parts3d, x2d, interpret=False):
    return pl.pallas_call(
        _reduce_node_stage,
        grid=(_ROWS // _BLK,),
        in_specs=[
            pl.BlockSpec(memory_space=pltpu.SMEM),
            pl.BlockSpec((N_TILES, _BLK, 128), lambda i: (0, i, 0)),
            pl.BlockSpec((_BLK, 128), lambda i: (i, 0)),
        ],
        out_specs=[
            pl.BlockSpec((_BLK, 128), lambda i: (i, 0)),
            pl.BlockSpec((_BLK, 128), lambda i: (i, 0)),
        ],
        out_shape=[
            jax.ShapeDtypeStruct((_ROWS, 128), jnp.float32),
            jax.ShapeDtypeStruct((_ROWS, 128), jnp.float32),
        ],
        interpret=interpret,
    )(wmat, parts3d, x2d)


def _reduce_add(p_ref, r_ref, o_ref):
    o_ref[...] = jnp.sum(p_ref[...], axis=0) + r_ref[...]


def _final_kernel(parts3d, r2d, interpret=False):
    return pl.pallas_call(
        _reduce_add,
        grid=(_ROWS // _BLK,),
        in_specs=[
            pl.BlockSpec((N_TILES, _BLK, 128), lambda i: (0, i, 0)),
            pl.BlockSpec((_BLK, 128), lambda i: (i, 0)),
        ],
        out_specs=pl.BlockSpec((_BLK, 128), lambda i: (i, 0)),
        out_shape=jax.ShapeDtypeStruct((_ROWS, 128), jnp.float32),
        interpret=interpret,
    )(parts3d, r2d)


@jax.jit
def _run(x, edge_index, edge_weight,
         W1_rel, b1_rel, W1_root, W2_rel, b2_rel, W2_root):
    x_flat = x.reshape(-1)
    x_pad2d = jnp.concatenate(
        [x_flat, jnp.zeros((N_PAD - N,), jnp.float32)]).reshape(_ROWS, 128)

    edge_pass = _make_edge_pass()

    _, s_parts = edge_pass(x_flat, edge_index, edge_weight)

    wmat = jnp.stack([
        W1_rel[:, 0], b1_rel, W1_root[:, 0],
        W2_rel[0, :], W2_root[0, :],
        jnp.full((H,), b2_rel[0], jnp.float32),
    ])
    t2d, r2d = _node_kernel(wmat, s_parts.reshape(N_TILES, _ROWS, 128),
                            x_pad2d)
    t_flat = t2d.reshape(-1)[:N]

    _, u_parts = edge_pass(t_flat, edge_index, edge_weight)
    out2d = _final_kernel(u_parts.reshape(N_TILES, _ROWS, 128), r2d)
    return out2d.reshape(-1)[:N]


def kernel(x, edge_index, edge_weight,
           W1_rel, b1_rel, W1_root, W2_rel, b2_rel, W2_root):
    return _run(x, edge_index, edge_weight,
                W1_rel, b1_rel, W1_root, W2_rel, b2_rel, W2_root)
